# baseline (speedup 1.0000x reference)
import functools

import jax
import jax.numpy as jnp
from jax import lax
from jax.experimental import pallas as pl
from jax.experimental.pallas import tpu as pltpu
from jax.experimental.pallas import tpu_sc as plsc

N = 19
L = 16
NEG = -1e30

OFF_X = 0
OFF_W1 = 32
OFF_AS1 = 96
OFF_AD1 = 160
OFF_B1 = 224
OFF_W2 = 288
OFF_AS2 = 8480
OFF_AD2 = 8608
OFF_B2 = 8736
OFF_WCR = 8864
OFF_WCM = 11296
OFF_WM = 11328
OFF_BM = 11360
OFF_SCAL = 11392
OFF_BC = 11408
TOT = 11424

S_AS2 = 0
S_ALPHA = 64


def _lrelu(t):
    return jnp.where(t >= 0.0, t, 0.2 * t)


def _body(x_hbm, mmse_hbm, w1_hbm, as1_hbm, ad1_hbm, b1_hbm, w2_hbm,
          as2_hbm, ad2_hbm, b2_hbm, wm_hbm, bm_hbm, wc_hbm, bc_hbm,
          out_hbm, buf, h_ref, m_ref, s_ref, out_v, sem):
    cid = lax.axis_index("c")
    sid = lax.axis_index("s")

    @pl.when(jnp.logical_and(cid == 0, sid == 0))
    def _():
        copies = [
            (x_hbm, pl.ds(OFF_X, N)),
            (mmse_hbm, pl.ds(OFF_SCAL, 1)),
            (w1_hbm, pl.ds(OFF_W1, 64)),
            (as1_hbm, pl.ds(OFF_AS1, 64)),
            (ad1_hbm, pl.ds(OFF_AD1, 64)),
            (b1_hbm, pl.ds(OFF_B1, 64)),
            (w2_hbm, pl.ds(OFF_W2, 8192)),
            (as2_hbm, pl.ds(OFF_AS2, 128)),
            (ad2_hbm, pl.ds(OFF_AD2, 128)),
            (b2_hbm, pl.ds(OFF_B2, 128)),
            (wm_hbm, pl.ds(OFF_WM, 32)),
            (bm_hbm, pl.ds(OFF_BM, 32)),
            (bc_hbm, pl.ds(OFF_BC, 1)),
        ]
        handles = [pltpu.async_copy(s, buf.at[d], sem) for s, d in copies]
        handles.append(
            pltpu.async_copy(wc_hbm.at[pl.ds(0, N * 128)],
                             buf.at[pl.ds(OFF_WCR, N * 128)], sem))
        handles.append(
            pltpu.async_copy(wc_hbm.at[pl.ds(N * 128, 32)],
                             buf.at[pl.ds(OFF_WCM, 32)], sem))
        for hnd in handles:
            hnd.wait()

        def vl(off):
            return buf[pl.ds(off, L)]

        lane0_iota = lax.iota(jnp.int32, L)
        xa = vl(OFF_X)
        xb = jnp.where(lane0_iota < (N - L), vl(OFF_X + L), 0.0)

        acc_s = vl(OFF_W1) * vl(OFF_AS1)
        acc_d = vl(OFF_W1) * vl(OFF_AD1)
        for k in range(1, 4):
            acc_s = acc_s + vl(OFF_W1 + 16 * k) * vl(OFF_AS1 + 16 * k)
            acc_d = acc_d + vl(OFF_W1 + 16 * k) * vl(OFF_AD1 + 16 * k)
        cs1 = jnp.sum(acc_s)
        cd1 = jnp.sum(acc_d)

        as1a = xa * cs1
        as1b = xb * cs1
        lane = lax.iota(jnp.int32, L)
        tail_mask = lane < (N - L)

        def bcast(ref, idx):
            return plsc.load_gather(ref, [jnp.full((L,), idx, jnp.int32)])

        lane0 = lane == 0

        ys = []
        for i in range(N):
            adi = (xa[i] if i < L else xb[i - L]) * cd1
            e_a = _lrelu(as1a + adi)
            e_b = jnp.where(tail_mask, _lrelu(as1b + adi), NEG)
            m = jnp.maximum(jnp.max(e_a), jnp.max(e_b))
            p_a = jnp.exp(e_a - m)
            p_b = jnp.exp(e_b - m)
            s = jnp.sum(p_a) + jnp.sum(p_b)
            num = jnp.sum(p_a * xa) + jnp.sum(p_b * xb)
            ys.append(jnp.broadcast_to(num, (L,)) / jnp.broadcast_to(s, (L,)))

        w1k = [vl(OFF_W1 + 16 * k) for k in range(4)]
        b1k = [vl(OFF_B1 + 16 * k) for k in range(4)]
        for i in range(N):
            for k in range(4):
                h_ref[pl.ds(i * 64 + 16 * k, L)] = jnp.maximum(
                    ys[i] * w1k[k] + b1k[k], 0.0)

        zero = jnp.zeros((L,), jnp.float32)

        def vsvd_step(c, carry):
            base = OFF_W2 + c * 64
            ss = bcast(buf, OFF_AS2 + c)
            sd = bcast(buf, OFF_AD2 + c)
            out = []
            for k in range(4):
                w = buf[pl.ds(base + 16 * k, L)]
                out.append(carry[k] + ss * w)
                out.append(carry[4 + k] + sd * w)
            return tuple(out[0::2]) + tuple(out[1::2])

        vsvd = lax.fori_loop(0, 128, vsvd_step, (zero,) * 8)
        vs2 = vsvd[:4]
        vd2 = vsvd[4:]

        s_ref[pl.ds(S_AS2 + 16, L)] = jnp.full((L,), NEG, jnp.float32)
        ad2 = []
        for i in range(N):
            hk = [h_ref[pl.ds(i * 64 + 16 * k, L)] for k in range(4)]
            a_s = hk[0] * vs2[0]
            a_d = hk[0] * vd2[0]
            for k in range(1, 4):
                a_s = a_s + hk[k] * vs2[k]
                a_d = a_d + hk[k] * vd2[k]
            plsc.store_scatter(
                s_ref, [jnp.full((L,), S_AS2 + i, jnp.int32)],
                jnp.broadcast_to(jnp.sum(a_s), (L,)), mask=lane0)
            ad2.append(jnp.sum(a_d))

        as2a = s_ref[pl.ds(S_AS2, L)]
        as2b = s_ref[pl.ds(S_AS2 + L, L)]

        for i in range(N):
            e_a = _lrelu(as2a + ad2[i])
            e_b = _lrelu(as2b + ad2[i])
            m = jnp.maximum(jnp.max(e_a), jnp.max(e_b))
            p_a = jnp.exp(e_a - m)
            p_b = jnp.exp(e_b - m)
            rs = 1.0 / jnp.broadcast_to(jnp.sum(p_a) + jnp.sum(p_b), (L,))
            s_ref[pl.ds(S_ALPHA, L)] = p_a
            s_ref[pl.ds(S_ALPHA + L, L)] = p_b

            def m_step(j, carry):
                pj = bcast(s_ref, S_ALPHA + j)
                return tuple(
                    carry[k] + pj * h_ref[pl.ds(j * 64 + 16 * k, L)]
                    for k in range(4))

            mk = lax.fori_loop(0, N, m_step, (zero,) * 4)
            for k in range(4):
                m_ref[pl.ds(i * 64 + 16 * k, L)] = mk[k] * rs

        p1v = zero
        for rows in (range(0, 8), range(8, 16), range(16, 19)):
            rows = list(rows)
            G = len(rows)

            def u_step(c, carry, rows=rows, G=G):
                base = OFF_W2 + c * 64
                wk = [buf[pl.ds(base + 16 * k, L)] for k in range(4)]
                out = list(carry)
                for r, i in enumerate(rows):
                    s = bcast(buf, OFF_WCR + i * 128 + c)
                    for k in range(4):
                        out[r * 4 + k] = out[r * 4 + k] + s * wk[k]
                return tuple(out)

            uacc = lax.fori_loop(0, 128, u_step, (zero,) * (4 * G))
            for r, i in enumerate(rows):
                for k in range(4):
                    p1v = p1v + uacc[r * 4 + k] * m_ref[pl.ds(i * 64 + 16 * k, L)]

        def col_step(i, carry):
            return tuple(
                carry[k] + buf[pl.ds(OFF_WCR + i * 128 + 16 * k, L)]
                for k in range(8))

        cols = lax.fori_loop(0, N, col_step, (zero,) * 8)
        for k in range(8):
            p1v = p1v + cols[k] * vl(OFF_B2 + 16 * k)

        scal = vl(OFF_SCAL)
        mmse = scal[0]
        for k in range(2):
            t = mmse * vl(OFF_WM + 16 * k) + vl(OFF_BM + 16 * k)
            p1v = p1v + t * vl(OFF_WCM + 16 * k)

        z = jnp.sum(p1v) + vl(OFF_BC)[0]
        zv = jnp.broadcast_to(z, (L,))
        out_v[...] = 1.0 / (1.0 + jnp.exp(-zv))
        pltpu.sync_copy(out_v, out_hbm)


@jax.jit
def _run(x, mmse, w1, as1, ad1, b1, w2, as2, ad2, b2, wm, bm, wc, bc):
    mesh = plsc.VectorSubcoreMesh(
        core_axis_name="c", subcore_axis_name="s", num_cores=2,
        num_subcores=16)
    f = pl.kernel(
        _body,
        out_type=jax.ShapeDtypeStruct((L,), jnp.float32),
        mesh=mesh,
        compiler_params=pltpu.CompilerParams(needs_layout_passes=False),
        scratch_types=[
            pltpu.VMEM((TOT,), jnp.float32),
            pltpu.VMEM((N * 64,), jnp.float32),
            pltpu.VMEM((N * 64,), jnp.float32),
            pltpu.VMEM((128,), jnp.float32),
            pltpu.VMEM((L,), jnp.float32),
            pltpu.SemaphoreType.DMA,
        ],
    )
    return f(x, mmse, w1, as1, ad1, b1, w2, as2, ad2, b2, wm, bm, wc, bc)


def kernel(eeg_dem_scores, mmse, W1, a_src1, a_dst1, b1, W2, a_src2,
           a_dst2, b2, Wm, bm, Wc, bc):
    out = _run(eeg_dem_scores.reshape(N), mmse, W1.reshape(64), a_src1,
               a_dst1, b1, W2.reshape(128 * 64), a_src2, a_dst2, b2,
               Wm.reshape(32), bm, Wc.reshape(2464), bc)
    return out[0:1].reshape(1, 1)


# speedup vs baseline: 1.0474x; 1.0474x over previous
import functools

import jax
import jax.numpy as jnp
from jax import lax
from jax.experimental import pallas as pl
from jax.experimental.pallas import tpu as pltpu
from jax.experimental.pallas import tpu_sc as plsc

N = 19
L = 16
NEG = -1e30

OFF_X = 0
OFF_W1 = 32
OFF_AS1 = 96
OFF_AD1 = 160
OFF_B1 = 224
OFF_W2 = 288
OFF_AS2 = 8480
OFF_AD2 = 8608
OFF_B2 = 8736
OFF_WCR = 8864
OFF_WCM = 11296
OFF_WM = 11328
OFF_BM = 11360
OFF_SCAL = 11392
OFF_BC = 11408
TOT = 11424

S_AS2 = 0
S_ALPHA = 64


def _lrelu(t):
    return jnp.where(t >= 0.0, t, 0.2 * t)


def _body(x_hbm, mmse_hbm, w1_hbm, as1_hbm, ad1_hbm, b1_hbm, w2_hbm,
          as2_hbm, ad2_hbm, b2_hbm, wm_hbm, bm_hbm, wc_hbm, bc_hbm,
          out_hbm, buf, h_ref, m_ref, s_ref, out_v, sem):
    cid = lax.axis_index("c")
    sid = lax.axis_index("s")

    @pl.when(jnp.logical_and(cid == 0, sid == 0))
    def _():
        copies = [
            (x_hbm, pl.ds(OFF_X, N)),
            (mmse_hbm, pl.ds(OFF_SCAL, 1)),
            (w1_hbm, pl.ds(OFF_W1, 64)),
            (as1_hbm, pl.ds(OFF_AS1, 64)),
            (ad1_hbm, pl.ds(OFF_AD1, 64)),
            (b1_hbm, pl.ds(OFF_B1, 64)),
            (w2_hbm, pl.ds(OFF_W2, 8192)),
            (as2_hbm, pl.ds(OFF_AS2, 128)),
            (ad2_hbm, pl.ds(OFF_AD2, 128)),
            (b2_hbm, pl.ds(OFF_B2, 128)),
            (wm_hbm, pl.ds(OFF_WM, 32)),
            (bm_hbm, pl.ds(OFF_BM, 32)),
            (bc_hbm, pl.ds(OFF_BC, 1)),
        ]
        handles = [pltpu.async_copy(s, buf.at[d], sem) for s, d in copies]
        handles.append(
            pltpu.async_copy(wc_hbm.at[pl.ds(0, N * 128)],
                             buf.at[pl.ds(OFF_WCR, N * 128)], sem))
        handles.append(
            pltpu.async_copy(wc_hbm.at[pl.ds(N * 128, 32)],
                             buf.at[pl.ds(OFF_WCM, 32)], sem))
        for hnd in handles:
            hnd.wait()

        def vl(off):
            return buf[pl.ds(off, L)]

        lane0_iota = lax.iota(jnp.int32, L)
        xa = vl(OFF_X)
        xb = jnp.where(lane0_iota < (N - L), vl(OFF_X + L), 0.0)

        acc_s = vl(OFF_W1) * vl(OFF_AS1)
        acc_d = vl(OFF_W1) * vl(OFF_AD1)
        for k in range(1, 4):
            acc_s = acc_s + vl(OFF_W1 + 16 * k) * vl(OFF_AS1 + 16 * k)
            acc_d = acc_d + vl(OFF_W1 + 16 * k) * vl(OFF_AD1 + 16 * k)
        cs1 = jnp.sum(acc_s)
        cd1 = jnp.sum(acc_d)

        as1a = xa * cs1
        as1b = xb * cs1
        lane = lax.iota(jnp.int32, L)
        tail_mask = lane < (N - L)

        def bcast(ref, idx):
            return plsc.load_gather(ref, [jnp.full((L,), idx, jnp.int32)])

        lane0 = lane == 0

        ys = []
        for i in range(N):
            adi = (xa[i] if i < L else xb[i - L]) * cd1
            e_a = _lrelu(as1a + adi)
            e_b = jnp.where(tail_mask, _lrelu(as1b + adi), NEG)
            m = jnp.maximum(jnp.max(e_a), jnp.max(e_b))
            p_a = jnp.exp(e_a - m)
            p_b = jnp.exp(e_b - m)
            s = jnp.sum(p_a) + jnp.sum(p_b)
            num = jnp.sum(p_a * xa) + jnp.sum(p_b * xb)
            ys.append(jnp.broadcast_to(num, (L,)) / jnp.broadcast_to(s, (L,)))

        w1k = [vl(OFF_W1 + 16 * k) for k in range(4)]
        b1k = [vl(OFF_B1 + 16 * k) for k in range(4)]
        for i in range(N):
            for k in range(4):
                h_ref[pl.ds(i * 64 + 16 * k, L)] = jnp.maximum(
                    ys[i] * w1k[k] + b1k[k], 0.0)

        zero = jnp.zeros((L,), jnp.float32)

        def vsvd_step(c, carry):
            base = OFF_W2 + c * 64
            ss = bcast(buf, OFF_AS2 + c)
            sd = bcast(buf, OFF_AD2 + c)
            out = []
            for k in range(4):
                w = buf[pl.ds(base + 16 * k, L)]
                out.append(carry[k] + ss * w)
                out.append(carry[4 + k] + sd * w)
            return tuple(out[0::2]) + tuple(out[1::2])

        vsvd = lax.fori_loop(0, 128, vsvd_step, (zero,) * 8)
        vs2 = vsvd[:4]
        vd2 = vsvd[4:]

        s_ref[pl.ds(S_AS2 + 16, L)] = jnp.full((L,), NEG, jnp.float32)
        ad2 = []
        for i in range(N):
            hk = [h_ref[pl.ds(i * 64 + 16 * k, L)] for k in range(4)]
            a_s = hk[0] * vs2[0]
            a_d = hk[0] * vd2[0]
            for k in range(1, 4):
                a_s = a_s + hk[k] * vs2[k]
                a_d = a_d + hk[k] * vd2[k]
            plsc.store_scatter(
                s_ref, [jnp.full((L,), S_AS2 + i, jnp.int32)],
                jnp.broadcast_to(jnp.sum(a_s), (L,)), mask=lane0)
            ad2.append(jnp.sum(a_d))

        as2a = s_ref[pl.ds(S_AS2, L)]
        as2b = s_ref[pl.ds(S_AS2 + L, L)]

        for i in range(N):
            e_a = _lrelu(as2a + ad2[i])
            e_b = _lrelu(as2b + ad2[i])
            m = jnp.maximum(jnp.max(e_a), jnp.max(e_b))
            p_a = jnp.exp(e_a - m)
            p_b = jnp.exp(e_b - m)
            rs = 1.0 / jnp.broadcast_to(jnp.sum(p_a) + jnp.sum(p_b), (L,))
            s_ref[pl.ds(S_ALPHA, L)] = p_a
            s_ref[pl.ds(S_ALPHA + L, L)] = p_b

            def m_step(j, carry):
                pj = bcast(s_ref, S_ALPHA + j)
                return tuple(
                    carry[k] + pj * h_ref[pl.ds(j * 64 + 16 * k, L)]
                    for k in range(4))

            mk = lax.fori_loop(0, N, m_step, (zero,) * 4)
            for k in range(4):
                m_ref[pl.ds(i * 64 + 16 * k, L)] = mk[k] * rs

        p1v = zero
        for rows in (range(0, 8), range(8, 16), range(16, 19)):
            rows = list(rows)
            G = len(rows)

            def u_step(c, carry, rows=rows, G=G):
                base = OFF_W2 + c * 64
                wk = [buf[pl.ds(base + 16 * k, L)] for k in range(4)]
                out = list(carry)
                for r, i in enumerate(rows):
                    s = bcast(buf, OFF_WCR + i * 128 + c)
                    for k in range(4):
                        out[r * 4 + k] = out[r * 4 + k] + s * wk[k]
                return tuple(out)

            uacc = lax.fori_loop(0, 128, u_step, (zero,) * (4 * G))
            for r, i in enumerate(rows):
                for k in range(4):
                    p1v = p1v + uacc[r * 4 + k] * m_ref[pl.ds(i * 64 + 16 * k, L)]

        def col_step(i, carry):
            return tuple(
                carry[k] + buf[pl.ds(OFF_WCR + i * 128 + 16 * k, L)]
                for k in range(8))

        cols = lax.fori_loop(0, N, col_step, (zero,) * 8)
        for k in range(8):
            p1v = p1v + cols[k] * vl(OFF_B2 + 16 * k)

        scal = vl(OFF_SCAL)
        mmse = scal[0]
        for k in range(2):
            t = mmse * vl(OFF_WM + 16 * k) + vl(OFF_BM + 16 * k)
            p1v = p1v + t * vl(OFF_WCM + 16 * k)

        z = jnp.sum(p1v) + vl(OFF_BC)[0]
        zv = jnp.broadcast_to(z, (L,))
        out_v[...] = 1.0 / (1.0 + jnp.exp(-zv))
        pltpu.sync_copy(out_v, out_hbm)


@jax.jit
def _run(x, mmse, w1, as1, ad1, b1, w2, as2, ad2, b2, wm, bm, wc, bc):
    mesh = plsc.VectorSubcoreMesh(
        core_axis_name="c", subcore_axis_name="s", num_cores=1,
        num_subcores=16)
    f = pl.kernel(
        _body,
        out_type=jax.ShapeDtypeStruct((L,), jnp.float32),
        mesh=mesh,
        compiler_params=pltpu.CompilerParams(needs_layout_passes=False),
        scratch_types=[
            pltpu.VMEM((TOT,), jnp.float32),
            pltpu.VMEM((N * 64,), jnp.float32),
            pltpu.VMEM((N * 64,), jnp.float32),
            pltpu.VMEM((128,), jnp.float32),
            pltpu.VMEM((L,), jnp.float32),
            pltpu.SemaphoreType.DMA,
        ],
    )
    return f(x, mmse, w1, as1, ad1, b1, w2, as2, ad2, b2, wm, bm, wc, bc)


def kernel(eeg_dem_scores, mmse, W1, a_src1, a_dst1, b1, W2, a_src2,
           a_dst2, b2, Wm, bm, Wc, bc):
    out = _run(eeg_dem_scores.reshape(N), mmse, W1.reshape(64), a_src1,
               a_dst1, b1, W2.reshape(128 * 64), a_src2, a_dst2, b2,
               Wm.reshape(32), bm, Wc.reshape(2464), bc)
    return out[0:1].reshape(1, 1)


# speedup vs baseline: 1.3964x; 1.3332x over previous
import jax
import jax.numpy as jnp
from jax import lax
from jax.experimental import pallas as pl
from jax.experimental.pallas import tpu as pltpu
from jax.experimental.pallas import tpu_sc as plsc

N = 19
L = 16
NEG = -1e30

OFF_X = 0
OFF_W1 = 32
OFF_AS1 = 96
OFF_AD1 = 160
OFF_B1 = 224
OFF_W2 = 288
OFF_AS2 = 8480
OFF_AD2 = 8608
OFF_B2 = 8736
OFF_WCR = 8864
OFF_WCM = 11296
OFF_WM = 11328
OFF_BM = 11360
OFF_SCAL = 11392
OFF_BC = 11408
TOT = 11424

S_AS2 = 0
S_AD2 = 32
S_ALPHA = 64
S_Y = 96
S_MX = 128
S_SS = 160
S_VS = 192
S_VD = 256


def _lrelu(t):
    return jnp.where(t >= 0.0, t, 0.2 * t)


def _body(x_hbm, mmse_hbm, w1_hbm, as1_hbm, ad1_hbm, b1_hbm, w2_hbm,
          as2_hbm, ad2_hbm, b2_hbm, wm_hbm, bm_hbm, wc_hbm, bc_hbm,
          out_hbm, buf, h_ref, ht_ref, s_ref, p_stage, out_v, shared, sem):
    tid = lax.axis_index("s")
    lane = lax.iota(jnp.int32, L)
    tail_mask = lane < (N - L)

    copies = [
        (x_hbm, pl.ds(OFF_X, N)),
        (mmse_hbm, pl.ds(OFF_SCAL, 1)),
        (w1_hbm, pl.ds(OFF_W1, 64)),
        (as1_hbm, pl.ds(OFF_AS1, 64)),
        (ad1_hbm, pl.ds(OFF_AD1, 64)),
        (b1_hbm, pl.ds(OFF_B1, 64)),
        (w2_hbm, pl.ds(OFF_W2, 8192)),
        (as2_hbm, pl.ds(OFF_AS2, 128)),
        (ad2_hbm, pl.ds(OFF_AD2, 128)),
        (b2_hbm, pl.ds(OFF_B2, 128)),
        (wm_hbm, pl.ds(OFF_WM, 32)),
        (bm_hbm, pl.ds(OFF_BM, 32)),
        (bc_hbm, pl.ds(OFF_BC, 1)),
    ]
    handles = [pltpu.async_copy(s, buf.at[d], sem) for s, d in copies]
    handles.append(
        pltpu.async_copy(wc_hbm.at[pl.ds(0, N * 128)],
                         buf.at[pl.ds(OFF_WCR, N * 128)], sem))
    handles.append(
        pltpu.async_copy(wc_hbm.at[pl.ds(N * 128, 32)],
                         buf.at[pl.ds(OFF_WCM, 32)], sem))
    for hnd in handles:
        hnd.wait()
    plsc.subcore_barrier()

    def vl(off):
        return buf[pl.ds(off, L)]

    def bcast(ref, idx):
        return plsc.load_gather(ref, [jnp.full((L,), idx, jnp.int32)])

    zero = jnp.zeros((L,), jnp.float32)
    negv = jnp.full((L,), NEG, jnp.float32)

    xa = vl(OFF_X)
    xb = jnp.where(tail_mask, vl(OFF_X + L), 0.0)

    acc_s = vl(OFF_W1) * vl(OFF_AS1)
    acc_d = vl(OFF_W1) * vl(OFF_AD1)
    for k in range(1, 4):
        acc_s = acc_s + vl(OFF_W1 + 16 * k) * vl(OFF_AS1 + 16 * k)
        acc_d = acc_d + vl(OFF_W1 + 16 * k) * vl(OFF_AD1 + 16 * k)
    cs1 = jnp.sum(acc_s)
    cd1 = jnp.sum(acc_d)

    ad1a = xa * cd1
    ad1b = xb * cd1

    def l1max_step(j, carry):
        mxa, mxb = carry
        asj = bcast(buf, OFF_X + j) * cs1
        return (jnp.maximum(mxa, _lrelu(ad1a + asj)),
                jnp.maximum(mxb, _lrelu(ad1b + asj)))

    mx1a, mx1b = lax.fori_loop(0, N, l1max_step, (negv, negv))

    def l1sum_step(j, carry):
        sa, sb, ya, yb = carry
        xj = bcast(buf, OFF_X + j)
        asj = xj * cs1
        p_a = jnp.exp(_lrelu(ad1a + asj) - mx1a)
        p_b = jnp.exp(_lrelu(ad1b + asj) - mx1b)
        return (sa + p_a, sb + p_b, ya + p_a * xj, yb + p_b * xj)

    sa, sb, ya, yb = lax.fori_loop(0, N, l1sum_step, (zero,) * 4)
    s_ref[pl.ds(S_Y, L)] = ya / sa
    s_ref[pl.ds(S_Y + L, L)] = yb / sb
    plsc.subcore_barrier()

    w1k = [vl(OFF_W1 + 16 * k) for k in range(4)]
    b1k = [vl(OFF_B1 + 16 * k) for k in range(4)]

    def h_step(i, carry):
        yv = bcast(s_ref, S_Y + i)
        for k in range(4):
            hk = jnp.maximum(yv * w1k[k] + b1k[k], 0.0)
            h_ref[pl.ds(i * 64 + 16 * k, L)] = hk
            plsc.store_scatter(ht_ref, [lane * 32 + (512 * k + i)], hk)
        return carry

    lax.fori_loop(0, N, h_step, 0)

    def vsvd_step(c, carry):
        base = OFF_W2 + c * 64
        ss = bcast(buf, OFF_AS2 + c)
        sd = bcast(buf, OFF_AD2 + c)
        out = []
        for k in range(4):
            w = buf[pl.ds(base + 16 * k, L)]
            out.append(carry[k] + ss * w)
            out.append(carry[4 + k] + sd * w)
        return tuple(out[0::2]) + tuple(out[1::2])

    vsvd = lax.fori_loop(0, 128, vsvd_step, (zero,) * 8, unroll=2)
    for k in range(4):
        s_ref[pl.ds(S_VS + 16 * k, L)] = vsvd[k]
        s_ref[pl.ds(S_VD + 16 * k, L)] = vsvd[4 + k]
    plsc.subcore_barrier()

    def as2_step(f, carry):
        a_sa, a_sb, a_da, a_db = carry
        vsf = bcast(s_ref, S_VS + f)
        vdf = bcast(s_ref, S_VD + f)
        hta = ht_ref[pl.ds(f * 32, L)]
        htb = ht_ref[pl.ds(f * 32 + L, L)]
        return (a_sa + vsf * hta, a_sb + vsf * htb,
                a_da + vdf * hta, a_db + vdf * htb)

    as2a, as2b, ad2a, ad2b = lax.fori_loop(0, 64, as2_step, (zero,) * 4,
                                           unroll=2)
    as2b = jnp.where(tail_mask, as2b, NEG)
    s_ref[pl.ds(S_AS2, L)] = as2a
    s_ref[pl.ds(S_AS2 + L, L)] = as2b
    s_ref[pl.ds(S_AD2, L)] = ad2a
    s_ref[pl.ds(S_AD2 + L, L)] = ad2b
    plsc.subcore_barrier()

    def l2max_step(j, carry):
        mxa, mxb = carry
        asj = bcast(s_ref, S_AS2 + j)
        return (jnp.maximum(mxa, _lrelu(ad2a + asj)),
                jnp.maximum(mxb, _lrelu(ad2b + asj)))

    mx2a, mx2b = lax.fori_loop(0, N, l2max_step, (negv, negv))

    def l2sum_step(j, carry):
        s2a, s2b = carry
        asj = bcast(s_ref, S_AS2 + j)
        return (s2a + jnp.exp(_lrelu(ad2a + asj) - mx2a),
                s2b + jnp.exp(_lrelu(ad2b + asj) - mx2b))

    s2a, s2b = lax.fori_loop(0, N, l2sum_step, (zero, zero))
    s_ref[pl.ds(S_MX, L)] = mx2a
    s_ref[pl.ds(S_MX + L, L)] = mx2b
    s_ref[pl.ds(S_SS, L)] = s2a
    s_ref[pl.ds(S_SS + L, L)] = s2b
    plsc.subcore_barrier()

    b2k = [vl(OFF_B2 + 16 * k) for k in range(8)]

    def row_work(i):
        ad2i = bcast(s_ref, S_AD2 + i)
        mi = bcast(s_ref, S_MX + i)
        rsi = 1.0 / bcast(s_ref, S_SS + i)
        p_a = jnp.exp(_lrelu(as2a + ad2i) - mi)
        p_b = jnp.exp(_lrelu(as2b + ad2i) - mi)
        s_ref[pl.ds(S_ALPHA, L)] = p_a
        s_ref[pl.ds(S_ALPHA + L, L)] = p_b
        plsc.subcore_barrier()

        def m_step(j, mc):
            pj = bcast(s_ref, S_ALPHA + j)
            return tuple(mc[k] + pj * h_ref[pl.ds(j * 64 + 16 * k, L)]
                         for k in range(4))

        mk = lax.fori_loop(0, N, m_step, (zero,) * 4)

        def u_step(c, uc):
            base = OFF_W2 + c * 64
            wc = bcast(buf, OFF_WCR + i * 128 + c)
            return tuple(uc[k] + wc * buf[pl.ds(base + 16 * k, L)]
                         for k in range(4))

        uk = lax.fori_loop(0, 128, u_step, (zero,) * 4, unroll=4)

        pv = zero
        for k in range(4):
            pv = pv + mk[k] * rsi * uk[k]
        for k in range(8):
            pv = pv + buf[pl.ds(OFF_WCR + i * 128 + 16 * k, L)] * b2k[k]
        return pv

    pv = row_work(tid)
    extra = tid < 3
    pv2 = row_work(jnp.where(extra, tid + L, tid))
    pv = pv + jnp.where(extra, pv2, zero)

    mmse = vl(OFF_SCAL)[0]
    head = zero
    for k in range(2):
        t = mmse * vl(OFF_WM + 16 * k) + vl(OFF_BM + 16 * k)
        head = head + t * vl(OFF_WCM + 16 * k)
    pv = pv + jnp.where(tid == 0, head, zero)

    out_v[...] = pv
    pltpu.sync_copy(out_v, shared.at[tid])
    plsc.subcore_barrier()

    @pl.when(tid == 0)
    def _fin():
        pltpu.sync_copy(shared, p_stage)

        def red_step(r, acc):
            return acc + p_stage[r, :]

        tot = lax.fori_loop(0, L, red_step, zero)
        z = jnp.sum(tot) + vl(OFF_BC)[0]
        zv = jnp.broadcast_to(z, (L,))
        out_v[...] = 1.0 / (1.0 + jnp.exp(-zv))
        pltpu.sync_copy(out_v, out_hbm)


@jax.jit
def _run(x, mmse, w1, as1, ad1, b1, w2, as2, ad2, b2, wm, bm, wc, bc):
    mesh = plsc.VectorSubcoreMesh(
        core_axis_name="c", subcore_axis_name="s", num_cores=1,
        num_subcores=16)
    f = pl.kernel(
        _body,
        out_type=jax.ShapeDtypeStruct((L,), jnp.float32),
        mesh=mesh,
        compiler_params=pltpu.CompilerParams(needs_layout_passes=False),
        scratch_types=[
            pltpu.VMEM((TOT,), jnp.float32),
            pltpu.VMEM((N * 64,), jnp.float32),
            pltpu.VMEM((64 * 32,), jnp.float32),
            pltpu.VMEM((352,), jnp.float32),
            pltpu.VMEM((L, L), jnp.float32),
            pltpu.VMEM((L,), jnp.float32),
            pltpu.VMEM_SHARED((L, L), jnp.float32),
            pltpu.SemaphoreType.DMA,
        ],
    )
    return f(x, mmse, w1, as1, ad1, b1, w2, as2, ad2, b2, wm, bm, wc, bc)


def kernel(eeg_dem_scores, mmse, W1, a_src1, a_dst1, b1, W2, a_src2,
           a_dst2, b2, Wm, bm, Wc, bc):
    out = _run(eeg_dem_scores.reshape(N), mmse, W1.reshape(64), a_src1,
               a_dst1, b1, W2.reshape(128 * 64), a_src2, a_dst2, b2,
               Wm.reshape(32), bm, Wc.reshape(2464), bc)
    return out[0:1].reshape(1, 1)
